# baseline (speedup 1.0000x reference)
import functools

import jax
import jax.numpy as jnp
from jax import lax
from jax.experimental import pallas as pl
from jax.experimental.pallas import tpu as pltpu
from jax.experimental.pallas import tpu_sc as plsc

_NUM_CORES = 2
_NUM_SUBCORES = 16
_LANES = 16
_PE_RING = 4
_X_RING = 6
_O_RING = 4


def _build_sc_call(n_rows: int, d_model: int, vocab: int):
    num_workers = _NUM_CORES * _NUM_SUBCORES
    n_per_w = n_rows // num_workers
    chunk = 8
    n_chunks = n_per_w // chunk

    mesh = plsc.VectorSubcoreMesh(
        core_axis_name="c",
        subcore_axis_name="s",
        num_cores=_NUM_CORES,
        num_subcores=_NUM_SUBCORES,
    )

    @functools.partial(
        pl.kernel,
        out_type=jax.ShapeDtypeStruct((n_rows, d_model), jnp.float32),
        mesh=mesh,
        scratch_types=[
            pltpu.VMEM((n_per_w,), jnp.int32),
            pltpu.VMEM((_PE_RING, chunk, d_model), jnp.float32),
            pltpu.VMEM((_X_RING, chunk, d_model), jnp.float32),
            pltpu.VMEM_SHARED(
                (_NUM_SUBCORES, _O_RING, chunk, d_model), jnp.float32
            ),
            pltpu.SemaphoreType.DMA((_PE_RING,)),
            pltpu.SemaphoreType.DMA((_X_RING,)),
            pltpu.SemaphoreType.DMA,
            pltpu.SemaphoreType.DMA((_O_RING,)),
        ],
    )
    def sc_add_pe(
        x_hbm,
        idx_hbm,
        pe_hbm,
        out_hbm,
        idx_v,
        pe_v,
        x_v,
        spo,
        gsem,
        lsem,
        dsem,
        ssem,
    ):
        cid = lax.axis_index("c")
        sid = lax.axis_index("s")
        wid = sid * _NUM_CORES + cid
        base = wid * n_per_w

        pltpu.sync_copy(idx_hbm.at[pl.ds(base, n_per_w)], idx_v)

        def gather(g):
            pb = lax.rem(g, _PE_RING)
            return pltpu.make_async_copy(
                pe_hbm.at[idx_v.at[pl.ds(g * chunk, chunk)]],
                pe_v.at[pb],
                gsem.at[pb],
            )

        def xload(g):
            xb = lax.rem(g, _X_RING)
            return pltpu.make_async_copy(
                x_hbm.at[pl.ds(base + g * chunk, chunk)],
                x_v.at[xb],
                lsem.at[xb],
            )

        def store_t2s(g):
            xb = lax.rem(g, _X_RING)
            ob = lax.rem(g, _O_RING)
            return pltpu.make_async_copy(
                x_v.at[xb], spo.at[sid, ob], dsem
            )

        def store_s2h(g):
            ob = lax.rem(g, _O_RING)
            return pltpu.make_async_copy(
                spo.at[sid, ob],
                out_hbm.at[pl.ds(base + g * chunk, chunk)],
                ssem.at[ob],
            )

        gather(0).start()
        xload(0).start()
        gather(1).start()
        xload(1).start()
        gather(2).start()
        xload(2).start()

        def chunk_body(g, carry):
            @pl.when(g >= 3)
            def _drain():
                store_s2h(g - 3).wait()

            @pl.when(g + 3 < n_chunks)
            def _prefetch():
                gather(g + 3).start()
                xload(g + 3).start()

            @pl.when(g >= 1)
            def _push():
                store_t2s(g - 1).wait()
                store_s2h(g - 1).start()

            gather(g).wait()
            xload(g).wait()

            pb = lax.rem(g, _PE_RING)
            xb = lax.rem(g, _X_RING)

            @plsc.parallel_loop(0, chunk, step=1, unroll=4)
            def _rows(r):
                for j in range(d_model // _LANES):
                    sl = pl.ds(j * _LANES, _LANES)
                    plsc.addupdate(x_v.at[xb, r, sl], pe_v[pb, r, sl])

            store_t2s(g).start()
            return carry

        lax.fori_loop(0, n_chunks, chunk_body, 0, unroll=False)
        store_t2s(n_chunks - 1).wait()
        store_s2h(n_chunks - 1).start()
        store_s2h(n_chunks - 3).wait()
        store_s2h(n_chunks - 2).wait()
        store_s2h(n_chunks - 1).wait()

    return sc_add_pe


def kernel(x, rand_idx, pe):
    b, s, d = x.shape
    n_rows = b * s
    vocab = pe.shape[1]

    x_flat = x.reshape(n_rows, d)
    idx_flat = rand_idx.reshape(n_rows).astype(jnp.int32)
    pe_flat = pe.reshape(vocab, d)

    out = _build_sc_call(n_rows, d, vocab)(x_flat, idx_flat, pe_flat)
    return out.reshape(1, b, s, d)


# speedup vs baseline: 1.1279x; 1.1279x over previous
import functools

import jax
import jax.numpy as jnp
from jax import lax
from jax.experimental import pallas as pl
from jax.experimental.pallas import tpu as pltpu
from jax.experimental.pallas import tpu_sc as plsc

_NUM_CORES = 2
_NUM_SUBCORES = 16
_LANES = 16
_PE_RING = 5
_X_RING = 8


def _build_sc_call(n_rows: int, d_model: int, vocab: int):
    num_workers = _NUM_CORES * _NUM_SUBCORES
    n_per_w = n_rows // num_workers
    chunk = 8
    n_chunks = n_per_w // chunk

    mesh = plsc.VectorSubcoreMesh(
        core_axis_name="c",
        subcore_axis_name="s",
        num_cores=_NUM_CORES,
        num_subcores=_NUM_SUBCORES,
    )

    @functools.partial(
        pl.kernel,
        out_type=jax.ShapeDtypeStruct((n_rows, d_model), jnp.float32),
        mesh=mesh,
        scratch_types=[
            pltpu.VMEM((n_per_w,), jnp.int32),
            pltpu.VMEM((_PE_RING, chunk, d_model), jnp.float32),
            pltpu.VMEM((_X_RING, chunk, d_model), jnp.float32),
            pltpu.SemaphoreType.DMA((_PE_RING,)),
            pltpu.SemaphoreType.DMA((_X_RING,)),
            pltpu.SemaphoreType.DMA((_X_RING,)),
        ],
    )
    def sc_add_pe(
        x_hbm, idx_hbm, pe_hbm, out_hbm, idx_v, pe_v, x_v, gsem, lsem, ssem
    ):
        cid = lax.axis_index("c")
        sid = lax.axis_index("s")
        wid = sid * _NUM_CORES + cid
        base = wid * n_per_w

        pltpu.sync_copy(idx_hbm.at[pl.ds(base, n_per_w)], idx_v)

        def gather(g):
            pb = lax.rem(g, _PE_RING)
            return pltpu.make_async_copy(
                pe_hbm.at[idx_v.at[pl.ds(g * chunk, chunk)]],
                pe_v.at[pb],
                gsem.at[pb],
            )

        def xload(g):
            xb = lax.rem(g, _X_RING)
            return pltpu.make_async_copy(
                x_hbm.at[pl.ds(base + g * chunk, chunk)],
                x_v.at[xb],
                lsem.at[xb],
            )

        def store(g):
            xb = lax.rem(g, _X_RING)
            return pltpu.make_async_copy(
                x_v.at[xb],
                out_hbm.at[pl.ds(base + g * chunk, chunk)],
                ssem.at[xb],
            )

        gather(0).start()
        xload(0).start()
        gather(1).start()
        xload(1).start()
        gather(2).start()
        xload(2).start()
        gather(3).start()
        xload(3).start()

        def chunk_body(g, carry):
            @pl.when(g >= 4)
            def _drain():
                store(g - 4).wait()

            @pl.when(g + 4 < n_chunks)
            def _prefetch():
                gather(g + 4).start()
                xload(g + 4).start()

            gather(g).wait()
            xload(g).wait()

            pb = lax.rem(g, _PE_RING)
            xb = lax.rem(g, _X_RING)

            @plsc.parallel_loop(0, chunk, step=1, unroll=4)
            def _rows(r):
                for j in range(d_model // _LANES):
                    sl = pl.ds(j * _LANES, _LANES)
                    plsc.addupdate(x_v.at[xb, r, sl], pe_v[pb, r, sl])

            store(g).start()
            return carry

        lax.fori_loop(0, n_chunks, chunk_body, 0, unroll=False)
        store(n_chunks - 4).wait()
        store(n_chunks - 3).wait()
        store(n_chunks - 2).wait()
        store(n_chunks - 1).wait()

    return sc_add_pe


def kernel(x, rand_idx, pe):
    b, s, d = x.shape
    n_rows = b * s
    vocab = pe.shape[1]

    x_flat = x.reshape(n_rows, d)
    idx_flat = rand_idx.reshape(n_rows).astype(jnp.int32)
    pe_flat = pe.reshape(vocab, d)

    out = _build_sc_call(n_rows, d, vocab)(x_flat, idx_flat, pe_flat)
    return out.reshape(1, b, s, d)
